# Initial kernel scaffold; baseline (speedup 1.0000x reference)
import jax
import jax.numpy as jnp
from jax.experimental import pallas as pl


def kernel(hidden_states, Wq, Wk, Wv, Wo, R):
    raise NotImplementedError("write your pallas kernel here")


# speedup vs baseline: 2.3847x; 2.3847x over previous
import functools

import jax
import jax.numpy as jnp
from jax.experimental import pallas as pl
from jax.experimental.pallas import tpu as pltpu

B, S, D, H = 2, 8192, 768, 12
DH = D // H
CHUNK = 128
NC = S // CHUNK
BH = B * H


def _attn_body(qi_ref, kip_ref, kic_ref, q_ref, kp_ref, kc_ref, vp_ref,
               vc_ref, o_ref):
    q = q_ref[0]
    k = jnp.concatenate([kp_ref[0], kc_ref[0]], axis=0)
    v = jnp.concatenate([vp_ref[0], vc_ref[0]], axis=0)
    dots = jax.lax.dot_general(q, k, (((1,), (1,)), ((), ())),
                               preferred_element_type=jnp.float32)
    qi = qi_ref[0]
    ki = jnp.concatenate([kip_ref[0], kic_ref[0]], axis=1)
    mask = (qi == ki).astype(jnp.float32)
    dots = dots - 1e5 * mask
    m = jnp.max(dots, axis=1, keepdims=True)
    p = jnp.exp(dots - m)
    s = jnp.sum(p, axis=1, keepdims=True)
    pv = jax.lax.dot_general(p, v, (((1,), (0,)), ((), ())),
                             preferred_element_type=jnp.float32)
    o_ref[0] = pv / s


def _chunked_attention(sq, sk, sv, qi, ki):
    qi3 = qi.reshape(BH, S, 1)
    ki3 = ki.reshape(BH * NC, 1, CHUNK)
    grid = (BH, NC)
    bspec_qkv = lambda imap: pl.BlockSpec((1, CHUNK, DH), imap)
    prev = lambda b, c: (b, (c + NC - 1) % NC, 0)
    cur = lambda b, c: (b, c, 0)
    kiprev = lambda b, c: (b * NC + (c + NC - 1) % NC, 0, 0)
    kicur = lambda b, c: (b * NC + c, 0, 0)
    return pl.pallas_call(
        _attn_body,
        grid=grid,
        in_specs=[
            pl.BlockSpec((1, CHUNK, 1), cur),
            pl.BlockSpec((1, 1, CHUNK), kiprev),
            pl.BlockSpec((1, 1, CHUNK), kicur),
            bspec_qkv(cur),
            bspec_qkv(prev),
            bspec_qkv(cur),
            bspec_qkv(prev),
            bspec_qkv(cur),
        ],
        out_specs=bspec_qkv(cur),
        out_shape=jax.ShapeDtypeStruct((BH, S, DH), jnp.float32),
    )(qi3, ki3, ki3, sq, sk, sk, sv, sv)


def _length_normalized(x, epsilon=1e-6):
    variance = jnp.mean(x ** 2, axis=-1, keepdims=True)
    return x / jnp.sqrt(variance + epsilon)


def kernel(hidden_states, Wq, Wk, Wv, Wo, R):
    q = jnp.einsum('bsd,dhf->bhsf', hidden_states, Wq)
    k = jnp.einsum('bsd,dhf->bhsf', hidden_states, Wk)
    v = jnp.einsum('bsd,dhf->bhsf', hidden_states, Wv)
    q = q.reshape(BH, S, DH)
    k = k.reshape(BH, S, DH)
    v = v.reshape(BH, S, DH)
    q_hash = jnp.einsum('nsf,f->ns', _length_normalized(q), R)
    k_hash = jnp.einsum('nsf,f->ns', _length_normalized(k), R)
    q_idx = jnp.argsort(q_hash, axis=-1).astype(jnp.int32)
    k_idx = jnp.argsort(k_hash, axis=-1).astype(jnp.int32)
    sq = jnp.take_along_axis(q, q_idx[..., None], axis=1)
    sk = jnp.take_along_axis(k, k_idx[..., None], axis=1) / jnp.sqrt(DH)
    sv = jnp.take_along_axis(v, k_idx[..., None], axis=1)
    so = _chunked_attention(sq, sk, sv, q_idx, k_idx)
    inv = jnp.argsort(q_idx, axis=-1)
    o = jnp.take_along_axis(so, inv[..., None], axis=1)
    o = o.reshape(B, H, S, DH)
    return jnp.einsum('bhsf,hfd->bsd', o, Wo)


# speedup vs baseline: 5.5807x; 2.3402x over previous
import functools

import jax
import jax.numpy as jnp
from jax import lax
from jax.experimental import pallas as pl
from jax.experimental.pallas import tpu as pltpu
from jax.experimental.pallas import tpu_sc as plsc

B, S, D, H = 2, 8192, 768, 12
DH = D // H
DP = 2 * DH
CHUNK = 128
NC = S // CHUNK
BH = B * H
NROWS = BH * S
NCHUNKS = NROWS // CHUNK
SC_CORES = 2
SC_SUBCORES = 16
NW = SC_CORES * SC_SUBCORES
CPW = NCHUNKS // NW


def _attn_body(qi_ref, kip_ref, kic_ref, q_ref, kp_ref, kc_ref, o_ref):
    q = q_ref[0]
    kv = jnp.concatenate([kp_ref[0], kc_ref[0]], axis=0)
    dots = jax.lax.dot_general(q, kv, (((1,), (1,)), ((), ())),
                               preferred_element_type=jnp.float32)
    dots = dots * (1.0 / 8.0)
    qi = qi_ref[0]
    ki = jnp.concatenate([kip_ref[0], kic_ref[0]], axis=1)
    mask = (qi == ki).astype(jnp.float32)
    dots = dots - 1e5 * mask
    m = jnp.max(dots, axis=1, keepdims=True)
    p = jnp.exp(dots - m)
    s = jnp.sum(p, axis=1, keepdims=True)
    v = kv[:, DH:]
    pv = jax.lax.dot_general(p, v, (((1,), (0,)), ((), ())),
                             preferred_element_type=jnp.float32)
    pv = pv / s
    o_ref[0] = jnp.concatenate(
        [pv, jnp.zeros((CHUNK, DH), jnp.float32)], axis=1)


def _chunked_attention(sq, skv, qi, ki):
    qi3 = qi.reshape(BH, S, 1)
    ki3 = ki.reshape(BH * NC, 1, CHUNK)
    grid = (BH, NC)
    bspec = lambda imap: pl.BlockSpec((1, CHUNK, DP), imap)
    prev = lambda b, c: (b, (c + NC - 1) % NC, 0)
    cur = lambda b, c: (b, c, 0)
    kiprev = lambda b, c: (b * NC + (c + NC - 1) % NC, 0, 0)
    kicur = lambda b, c: (b * NC + c, 0, 0)
    return pl.pallas_call(
        _attn_body,
        grid=grid,
        in_specs=[
            pl.BlockSpec((1, CHUNK, 1), cur),
            pl.BlockSpec((1, 1, CHUNK), kiprev),
            pl.BlockSpec((1, 1, CHUNK), kicur),
            bspec(cur),
            bspec(prev),
            bspec(cur),
        ],
        out_specs=bspec(cur),
        out_shape=jax.ShapeDtypeStruct((BH, S, DP), jnp.float32),
    )(qi3, ki3, ki3, sq, skv, skv)


_SC_MESH = plsc.VectorSubcoreMesh(
    core_axis_name="c", subcore_axis_name="s",
    num_cores=SC_CORES, num_subcores=SC_SUBCORES)


def _worker_id():
    return lax.axis_index("s") * SC_CORES + lax.axis_index("c")


@functools.partial(
    pl.kernel, mesh=_SC_MESH,
    out_type=[jax.ShapeDtypeStruct((NROWS, DP), jnp.float32)] * 2,
    scratch_types=[
        pltpu.VMEM((CHUNK,), jnp.int32),
        pltpu.VMEM((CHUNK, DP), jnp.float32),
        pltpu.SemaphoreType.DMA,
    ],
)
def _sc_gather(q_hbm, kv_hbm, qi_hbm, ki_hbm,
               sq_hbm, skv_hbm, idx_v, rows_v, sem):
    wid = _worker_id()
    for src, idx, dst in ((q_hbm, qi_hbm, sq_hbm),
                          (kv_hbm, ki_hbm, skv_hbm)):
        def body(i, _, src=src, idx=idx, dst=dst):
            base = (wid * CPW + i) * CHUNK
            pltpu.sync_copy(idx.at[pl.ds(base, CHUNK)], idx_v)
            pltpu.async_copy(src.at[idx_v], rows_v, sem).wait()
            pltpu.sync_copy(rows_v, dst.at[pl.ds(base, CHUNK)])
            return ()
        lax.fori_loop(0, CPW, body, ())


@functools.partial(
    pl.kernel, mesh=_SC_MESH,
    out_type=jax.ShapeDtypeStruct((NROWS, DP), jnp.float32),
    scratch_types=[
        pltpu.VMEM((CHUNK,), jnp.int32),
        pltpu.VMEM((CHUNK, DP), jnp.float32),
        pltpu.SemaphoreType.DMA,
    ],
)
def _sc_scatter(so_hbm, qi_hbm, o_hbm, idx_v, rows_v, sem):
    wid = _worker_id()

    def body(i, _):
        base = (wid * CPW + i) * CHUNK
        pltpu.sync_copy(qi_hbm.at[pl.ds(base, CHUNK)], idx_v)
        pltpu.sync_copy(so_hbm.at[pl.ds(base, CHUNK)], rows_v)
        pltpu.async_copy(rows_v, o_hbm.at[idx_v], sem).wait()
        return ()
    lax.fori_loop(0, CPW, body, ())


def _length_normalized(x, epsilon=1e-6):
    variance = jnp.mean(x ** 2, axis=-1, keepdims=True)
    return x / jnp.sqrt(variance + epsilon)


def kernel(hidden_states, Wq, Wk, Wv, Wo, R):
    q64 = jnp.einsum('bsd,dhf->bhsf', hidden_states, Wq).reshape(BH, S, DH)
    k64 = jnp.einsum('bsd,dhf->bhsf', hidden_states, Wk).reshape(BH, S, DH)
    v64 = jnp.einsum('bsd,dhf->bhsf', hidden_states, Wv).reshape(BH, S, DH)
    q_hash = jnp.einsum('nsf,f->ns', _length_normalized(q64), R)
    k_hash = jnp.einsum('nsf,f->ns', _length_normalized(k64), R)
    q = jnp.concatenate([q64, jnp.zeros_like(q64)], axis=-1)
    kv = jnp.concatenate([k64, v64], axis=-1)

    offs = jnp.arange(BH, dtype=jnp.int32)[:, None] * S
    q_idx = jnp.argsort(q_hash, axis=-1).astype(jnp.int32) + offs
    k_idx = jnp.argsort(k_hash, axis=-1).astype(jnp.int32) + offs
    qif = q_idx.reshape(NROWS)
    kif = k_idx.reshape(NROWS)

    sq, skv = _sc_gather(q.reshape(NROWS, DP), kv.reshape(NROWS, DP),
                         qif, kif)
    so = _chunked_attention(sq.reshape(BH, S, DP), skv.reshape(BH, S, DP),
                            q_idx, k_idx)
    o = _sc_scatter(so.reshape(NROWS, DP), qif)
    o = o.reshape(B, H, S, DP)
    Woz = jnp.concatenate([Wo, jnp.zeros_like(Wo)], axis=1)
    return jnp.einsum('bhsf,hfd->bsd', o, Woz)


# speedup vs baseline: 5.9138x; 1.0597x over previous
import functools

import jax
import jax.numpy as jnp
from jax import lax
from jax.experimental import pallas as pl
from jax.experimental.pallas import tpu as pltpu
from jax.experimental.pallas import tpu_sc as plsc

B, S, D, H = 2, 8192, 768, 12
DH = D // H
DP = 2 * DH
CHUNK = 128
NC = S // CHUNK
BH = B * H
NROWS = BH * S
NCHUNKS = NROWS // CHUNK
SC_CORES = 2
SC_SUBCORES = 16
NW = SC_CORES * SC_SUBCORES
CPW = NCHUNKS // NW


def _attn_body(qi_ref, kip_ref, kic_ref, q_ref, kp_ref, kc_ref, o_ref):
    q = q_ref[0]
    kv = jnp.concatenate([kp_ref[0], kc_ref[0]], axis=0)
    dots = jax.lax.dot_general(q, kv, (((1,), (1,)), ((), ())),
                               preferred_element_type=jnp.float32)
    dots = dots * (1.0 / 8.0)
    qi = qi_ref[0]
    ki = jnp.concatenate([kip_ref[0], kic_ref[0]], axis=1)
    mask = (qi == ki).astype(jnp.float32)
    dots = dots - 1e5 * mask
    m = jnp.max(dots, axis=1, keepdims=True)
    p = jnp.exp(dots - m)
    s = jnp.sum(p, axis=1, keepdims=True)
    v = kv[:, DH:]
    pv = jax.lax.dot_general(p, v, (((1,), (0,)), ((), ())),
                             preferred_element_type=jnp.float32)
    pv = pv / s
    o_ref[0] = jnp.concatenate(
        [pv, jnp.zeros((CHUNK, DH), jnp.float32)], axis=1)


def _chunked_attention(sq, skv, qi, ki):
    qi3 = qi.reshape(BH, S, 1)
    ki3 = ki.reshape(BH * NC, 1, CHUNK)
    grid = (BH, NC)
    bspec = lambda imap: pl.BlockSpec((1, CHUNK, DP), imap)
    prev = lambda b, c: (b, (c + NC - 1) % NC, 0)
    cur = lambda b, c: (b, c, 0)
    kiprev = lambda b, c: (b * NC + (c + NC - 1) % NC, 0, 0)
    kicur = lambda b, c: (b * NC + c, 0, 0)
    return pl.pallas_call(
        _attn_body,
        grid=grid,
        in_specs=[
            pl.BlockSpec((1, CHUNK, 1), cur),
            pl.BlockSpec((1, 1, CHUNK), kiprev),
            pl.BlockSpec((1, 1, CHUNK), kicur),
            bspec(cur),
            bspec(prev),
            bspec(cur),
        ],
        out_specs=bspec(cur),
        out_shape=jax.ShapeDtypeStruct((BH, S, DP), jnp.float32),
    )(qi3, ki3, ki3, sq, skv, skv)


_SC_MESH = plsc.VectorSubcoreMesh(
    core_axis_name="c", subcore_axis_name="s",
    num_cores=SC_CORES, num_subcores=SC_SUBCORES)


def _worker_id():
    return lax.axis_index("s") * SC_CORES + lax.axis_index("c")


@functools.partial(
    pl.kernel, mesh=_SC_MESH,
    out_type=[jax.ShapeDtypeStruct((NROWS, DP), jnp.float32)] * 2,
    scratch_types=[
        pltpu.VMEM((CPW, CHUNK), jnp.int32),
        pltpu.VMEM((CHUNK, DP), jnp.float32),
        pltpu.VMEM((CHUNK, DP), jnp.float32),
        pltpu.SemaphoreType.DMA,
        pltpu.SemaphoreType.DMA,
    ],
)
def _sc_gather(q_hbm, kv_hbm, qi_hbm, ki_hbm,
               sq_hbm, skv_hbm, idx_v, rows_a, rows_b, sem_a, sem_b):
    wid = _worker_id()
    for src, idx, dst in ((q_hbm, qi_hbm, sq_hbm),
                          (kv_hbm, ki_hbm, skv_hbm)):
        pltpu.sync_copy(idx.at[pl.ds(wid * CPW, CPW)], idx_v)

        def desc(c, buf, sem, src=src):
            return pltpu.make_async_copy(src.at[idx_v.at[c]], buf, sem)

        def store(c, buf, dst=dst):
            pltpu.sync_copy(buf, dst.at[pl.ds((wid * CPW + c) * CHUNK,
                                              CHUNK)])

        desc(0, rows_a, sem_a).start()

        def body(j, _):
            c0 = 2 * j
            desc(c0 + 1, rows_b, sem_b).start()
            desc(c0, rows_a, sem_a).wait()
            store(c0, rows_a)
            desc(c0 + 2, rows_a, sem_a).start()
            desc(c0 + 1, rows_b, sem_b).wait()
            store(c0 + 1, rows_b)
            return ()
        lax.fori_loop(0, CPW // 2 - 1, body, ())
        desc(CPW - 1, rows_b, sem_b).start()
        desc(CPW - 2, rows_a, sem_a).wait()
        store(CPW - 2, rows_a)
        desc(CPW - 1, rows_b, sem_b).wait()
        store(CPW - 1, rows_b)


@functools.partial(
    pl.kernel, mesh=_SC_MESH,
    out_type=jax.ShapeDtypeStruct((NROWS, DP), jnp.float32),
    scratch_types=[
        pltpu.VMEM((CPW, CHUNK), jnp.int32),
        pltpu.VMEM((CHUNK, DP), jnp.float32),
        pltpu.VMEM((CHUNK, DP), jnp.float32),
        pltpu.SemaphoreType.DMA,
        pltpu.SemaphoreType.DMA,
    ],
)
def _sc_scatter(so_hbm, qi_hbm, o_hbm, idx_v, rows_a, rows_b, sem_a, sem_b):
    wid = _worker_id()
    pltpu.sync_copy(qi_hbm.at[pl.ds(wid * CPW, CPW)], idx_v)

    def load(c, buf):
        pltpu.sync_copy(so_hbm.at[pl.ds((wid * CPW + c) * CHUNK, CHUNK)],
                        buf)

    def desc(c, buf, sem):
        return pltpu.make_async_copy(buf, o_hbm.at[idx_v.at[c]], sem)

    load(0, rows_a)
    desc(0, rows_a, sem_a).start()

    def body(j, _):
        c0 = 2 * j
        load(c0 + 1, rows_b)
        desc(c0 + 1, rows_b, sem_b).start()
        desc(c0, rows_a, sem_a).wait()
        load(c0 + 2, rows_a)
        desc(c0 + 2, rows_a, sem_a).start()
        desc(c0 + 1, rows_b, sem_b).wait()
        return ()
    lax.fori_loop(0, CPW // 2 - 1, body, ())
    load(CPW - 1, rows_b)
    desc(CPW - 1, rows_b, sem_b).start()
    desc(CPW - 2, rows_a, sem_a).wait()
    desc(CPW - 1, rows_b, sem_b).wait()


def _length_normalized(x, epsilon=1e-6):
    variance = jnp.mean(x ** 2, axis=-1, keepdims=True)
    return x / jnp.sqrt(variance + epsilon)


def kernel(hidden_states, Wq, Wk, Wv, Wo, R):
    q64 = jnp.einsum('bsd,dhf->bhsf', hidden_states, Wq).reshape(BH, S, DH)
    k64 = jnp.einsum('bsd,dhf->bhsf', hidden_states, Wk).reshape(BH, S, DH)
    v64 = jnp.einsum('bsd,dhf->bhsf', hidden_states, Wv).reshape(BH, S, DH)
    q_hash = jnp.einsum('nsf,f->ns', _length_normalized(q64), R)
    k_hash = jnp.einsum('nsf,f->ns', _length_normalized(k64), R)
    q = jnp.concatenate([q64, jnp.zeros_like(q64)], axis=-1)
    kv = jnp.concatenate([k64, v64], axis=-1)

    offs = jnp.arange(BH, dtype=jnp.int32)[:, None] * S
    q_idx = jnp.argsort(q_hash, axis=-1).astype(jnp.int32) + offs
    k_idx = jnp.argsort(k_hash, axis=-1).astype(jnp.int32) + offs
    qif = q_idx.reshape(NCHUNKS, CHUNK)
    kif = k_idx.reshape(NCHUNKS, CHUNK)

    sq, skv = _sc_gather(q.reshape(NROWS, DP), kv.reshape(NROWS, DP),
                         qif, kif)
    so = _chunked_attention(sq.reshape(BH, S, DP), skv.reshape(BH, S, DP),
                            q_idx, k_idx)
    o = _sc_scatter(so.reshape(NROWS, DP), qif)
    o = o.reshape(B, H, S, DP)
    Woz = jnp.concatenate([Wo, jnp.zeros_like(Wo)], axis=1)
    return jnp.einsum('bhsf,hfd->bsd', o, Woz)


# speedup vs baseline: 8.6114x; 1.4561x over previous
import functools

import jax
import jax.numpy as jnp
from jax import lax
from jax.experimental import pallas as pl
from jax.experimental.pallas import tpu as pltpu
from jax.experimental.pallas import tpu_sc as plsc

B, S, D, H = 2, 8192, 768, 12
DH = D // H
DP = 2 * DH
CHUNK = 128
NC = S // CHUNK
BH = B * H
NROWS = BH * S
NCHUNKS = NROWS // CHUNK
SC_CORES = 2
SC_SUBCORES = 16
NW = SC_CORES * SC_SUBCORES
CPW = NCHUNKS // NW


G = 4
GC = G * CHUNK
GN = NC // G


def _attn_body(qi_ref, kip_ref, kic_ref, q_ref, kp_ref, kc_ref, o_ref):
    kvall = jnp.concatenate([kp_ref[0], kc_ref[0]], axis=0)
    kiall = jnp.concatenate([kip_ref[0], kic_ref[0]], axis=1)
    for i in range(G):
        q = q_ref[0, i * CHUNK:(i + 1) * CHUNK]
        kv = kvall[(G - 1 + i) * CHUNK:(G + 1 + i) * CHUNK]
        ki = kiall[:, (G - 1 + i) * CHUNK:(G + 1 + i) * CHUNK]
        qi = qi_ref[0, i * CHUNK:(i + 1) * CHUNK]
        dots = jax.lax.dot_general(q, kv, (((1,), (1,)), ((), ())),
                                   preferred_element_type=jnp.float32)
        dots = dots * (1.0 / 8.0)
        mask = (qi == ki).astype(jnp.float32)
        dots = dots - 1e5 * mask
        m = jnp.max(dots, axis=1, keepdims=True)
        p = jnp.exp(dots - m)
        s = jnp.sum(p, axis=1, keepdims=True)
        v = kv[:, DH:]
        pv = jax.lax.dot_general(p, v, (((1,), (0,)), ((), ())),
                                 preferred_element_type=jnp.float32)
        pv = pv / s
        o_ref[0, i * CHUNK:(i + 1) * CHUNK] = jnp.concatenate(
            [pv, jnp.zeros((CHUNK, DH), jnp.float32)], axis=1)


def _chunked_attention(sq, skv, qi, ki):
    qi3 = qi.reshape(BH, S, 1)
    ki3 = ki.reshape(BH * GN, 1, GC)
    grid = (BH, GN)
    bspec = lambda imap: pl.BlockSpec((1, GC, DP), imap)
    prev = lambda b, g: (b, (g + GN - 1) % GN, 0)
    cur = lambda b, g: (b, g, 0)
    kiprev = lambda b, g: (b * GN + (g + GN - 1) % GN, 0, 0)
    kicur = lambda b, g: (b * GN + g, 0, 0)
    return pl.pallas_call(
        _attn_body,
        grid=grid,
        in_specs=[
            pl.BlockSpec((1, GC, 1), cur),
            pl.BlockSpec((1, 1, GC), kiprev),
            pl.BlockSpec((1, 1, GC), kicur),
            bspec(cur),
            bspec(prev),
            bspec(cur),
        ],
        out_specs=bspec(cur),
        out_shape=jax.ShapeDtypeStruct((BH, S, DP), jnp.float32),
    )(qi3, ki3, ki3, sq, skv, skv)


_SC_MESH = plsc.VectorSubcoreMesh(
    core_axis_name="c", subcore_axis_name="s",
    num_cores=SC_CORES, num_subcores=SC_SUBCORES)


def _worker_id():
    return lax.axis_index("s") * SC_CORES + lax.axis_index("c")


@functools.partial(
    pl.kernel, mesh=_SC_MESH,
    out_type=[jax.ShapeDtypeStruct((NROWS, DP), jnp.float32)] * 2,
    scratch_types=[
        pltpu.VMEM((CPW, CHUNK), jnp.int32),
        pltpu.VMEM((CHUNK, DP), jnp.float32),
        pltpu.VMEM((CHUNK, DP), jnp.float32),
        pltpu.SemaphoreType.DMA,
        pltpu.SemaphoreType.DMA,
    ],
)
def _sc_gather(q_hbm, kv_hbm, qi_hbm, ki_hbm,
               sq_hbm, skv_hbm, idx_v, rows_a, rows_b, sem_a, sem_b):
    wid = _worker_id()
    for src, idx, dst in ((q_hbm, qi_hbm, sq_hbm),
                          (kv_hbm, ki_hbm, skv_hbm)):
        pltpu.sync_copy(idx.at[pl.ds(wid * CPW, CPW)], idx_v)

        def desc(c, buf, sem, src=src):
            return pltpu.make_async_copy(src.at[idx_v.at[c]], buf, sem)

        def store(c, buf, dst=dst):
            pltpu.sync_copy(buf, dst.at[pl.ds((wid * CPW + c) * CHUNK,
                                              CHUNK)])

        desc(0, rows_a, sem_a).start()

        def body(j, _):
            c0 = 2 * j
            desc(c0 + 1, rows_b, sem_b).start()
            desc(c0, rows_a, sem_a).wait()
            store(c0, rows_a)
            desc(c0 + 2, rows_a, sem_a).start()
            desc(c0 + 1, rows_b, sem_b).wait()
            store(c0 + 1, rows_b)
            return ()
        lax.fori_loop(0, CPW // 2 - 1, body, ())
        desc(CPW - 1, rows_b, sem_b).start()
        desc(CPW - 2, rows_a, sem_a).wait()
        store(CPW - 2, rows_a)
        desc(CPW - 1, rows_b, sem_b).wait()
        store(CPW - 1, rows_b)


@functools.partial(
    pl.kernel, mesh=_SC_MESH,
    out_type=jax.ShapeDtypeStruct((NROWS, DP), jnp.float32),
    scratch_types=[
        pltpu.VMEM((CPW, CHUNK), jnp.int32),
        pltpu.VMEM((CHUNK, DP), jnp.float32),
        pltpu.VMEM((CHUNK, DP), jnp.float32),
        pltpu.SemaphoreType.DMA,
        pltpu.SemaphoreType.DMA,
    ],
)
def _sc_scatter(so_hbm, qi_hbm, o_hbm, idx_v, rows_a, rows_b, sem_a, sem_b):
    wid = _worker_id()
    pltpu.sync_copy(qi_hbm.at[pl.ds(wid * CPW, CPW)], idx_v)

    def load(c, buf):
        pltpu.sync_copy(so_hbm.at[pl.ds((wid * CPW + c) * CHUNK, CHUNK)],
                        buf)

    def desc(c, buf, sem):
        return pltpu.make_async_copy(buf, o_hbm.at[idx_v.at[c]], sem)

    load(0, rows_a)
    desc(0, rows_a, sem_a).start()

    def body(j, _):
        c0 = 2 * j
        load(c0 + 1, rows_b)
        desc(c0 + 1, rows_b, sem_b).start()
        desc(c0, rows_a, sem_a).wait()
        load(c0 + 2, rows_a)
        desc(c0 + 2, rows_a, sem_a).start()
        desc(c0 + 1, rows_b, sem_b).wait()
        return ()
    lax.fori_loop(0, CPW // 2 - 1, body, ())
    load(CPW - 1, rows_b)
    desc(CPW - 1, rows_b, sem_b).start()
    desc(CPW - 2, rows_a, sem_a).wait()
    desc(CPW - 1, rows_b, sem_b).wait()


def _length_normalized(x, epsilon=1e-6):
    variance = jnp.mean(x ** 2, axis=-1, keepdims=True)
    return x / jnp.sqrt(variance + epsilon)


def kernel(hidden_states, Wq, Wk, Wv, Wo, R):
    q64 = jnp.einsum('bsd,dhf->bhsf', hidden_states, Wq).reshape(BH, S, DH)
    k64 = jnp.einsum('bsd,dhf->bhsf', hidden_states, Wk).reshape(BH, S, DH)
    v64 = jnp.einsum('bsd,dhf->bhsf', hidden_states, Wv).reshape(BH, S, DH)
    q_hash = jnp.einsum('nsf,f->ns', _length_normalized(q64), R)
    k_hash = jnp.einsum('nsf,f->ns', _length_normalized(k64), R)
    q = jnp.concatenate([q64, jnp.zeros_like(q64)], axis=-1)
    kv = jnp.concatenate([k64, v64], axis=-1)

    offs = jnp.arange(BH, dtype=jnp.int32)[:, None] * S
    q_idx = jnp.argsort(q_hash, axis=-1).astype(jnp.int32) + offs
    k_idx = jnp.argsort(k_hash, axis=-1).astype(jnp.int32) + offs
    qif = q_idx.reshape(NCHUNKS, CHUNK)
    kif = k_idx.reshape(NCHUNKS, CHUNK)

    sq, skv = _sc_gather(q.reshape(NROWS, DP), kv.reshape(NROWS, DP),
                         qif, kif)
    so = _chunked_attention(sq.reshape(BH, S, DP), skv.reshape(BH, S, DP),
                            q_idx, k_idx)
    o = _sc_scatter(so.reshape(NROWS, DP), qif)
    o = o.reshape(B, H, S, DP)
    Woz = jnp.concatenate([Wo, jnp.zeros_like(Wo)], axis=1)
    return jnp.einsum('bhsf,hfd->bsd', o, Woz)


# speedup vs baseline: 9.2906x; 1.0789x over previous
import functools

import jax
import jax.numpy as jnp
from jax import lax
from jax.experimental import pallas as pl
from jax.experimental.pallas import tpu as pltpu
from jax.experimental.pallas import tpu_sc as plsc

B, S, D, H = 2, 8192, 768, 12
DH = D // H
DP = 2 * DH
CHUNK = 128
NC = S // CHUNK
BH = B * H
NROWS = BH * S
NCHUNKS = NROWS // CHUNK
SC_CORES = 2
SC_SUBCORES = 16
NW = SC_CORES * SC_SUBCORES
CPW = NCHUNKS // NW


G = 8
GC = G * CHUNK
GN = NC // G


def _attn_body(qi_ref, kip_ref, kic_ref, q_ref, kp_ref, kc_ref, o_ref):
    kvall = jnp.concatenate([kp_ref[0], kc_ref[0]], axis=0)
    kiall = jnp.concatenate([kip_ref[0], kic_ref[0]], axis=1)
    for i in range(G):
        q = q_ref[0, i * CHUNK:(i + 1) * CHUNK]
        kv = kvall[(G - 1 + i) * CHUNK:(G + 1 + i) * CHUNK]
        ki = kiall[:, (G - 1 + i) * CHUNK:(G + 1 + i) * CHUNK]
        qi = qi_ref[0, i * CHUNK:(i + 1) * CHUNK]
        dots = jax.lax.dot_general(q, kv, (((1,), (1,)), ((), ())),
                                   preferred_element_type=jnp.float32)
        dots = dots * (1.0 / 8.0)
        mask = (qi == ki).astype(jnp.float32)
        dots = dots - 1e5 * mask
        m = jnp.max(dots, axis=1, keepdims=True)
        p = jnp.exp(dots - m)
        s = jnp.sum(p, axis=1, keepdims=True)
        v = kv[:, DH:]
        pv = jax.lax.dot_general(p, v, (((1,), (0,)), ((), ())),
                                 preferred_element_type=jnp.float32)
        pv = pv / s
        o_ref[0, i * CHUNK:(i + 1) * CHUNK] = jnp.concatenate(
            [pv, jnp.zeros((CHUNK, DH), jnp.float32)], axis=1)


def _chunked_attention(sq, skv, qi, ki):
    qi3 = qi.reshape(BH, S, 1)
    ki3 = ki.reshape(BH * GN, 1, GC)
    grid = (BH, GN)
    bspec = lambda imap: pl.BlockSpec((1, GC, DP), imap)
    prev = lambda b, g: (b, (g + GN - 1) % GN, 0)
    cur = lambda b, g: (b, g, 0)
    kiprev = lambda b, g: (b * GN + (g + GN - 1) % GN, 0, 0)
    kicur = lambda b, g: (b * GN + g, 0, 0)
    return pl.pallas_call(
        _attn_body,
        grid=grid,
        in_specs=[
            pl.BlockSpec((1, GC, 1), cur),
            pl.BlockSpec((1, 1, GC), kiprev),
            pl.BlockSpec((1, 1, GC), kicur),
            bspec(cur),
            bspec(prev),
            bspec(cur),
        ],
        out_specs=bspec(cur),
        out_shape=jax.ShapeDtypeStruct((BH, S, DP), jnp.float32),
    )(qi3, ki3, ki3, sq, skv, skv)


_SC_MESH = plsc.VectorSubcoreMesh(
    core_axis_name="c", subcore_axis_name="s",
    num_cores=SC_CORES, num_subcores=SC_SUBCORES)


def _worker_id():
    return lax.axis_index("s") * SC_CORES + lax.axis_index("c")


@functools.partial(
    pl.kernel, mesh=_SC_MESH,
    out_type=[jax.ShapeDtypeStruct((NROWS, DP), jnp.float32)] * 2,
    scratch_types=[
        pltpu.VMEM((CPW, CHUNK), jnp.int32),
        pltpu.VMEM((CHUNK, DP), jnp.float32),
        pltpu.VMEM((CHUNK, DP), jnp.float32),
        pltpu.SemaphoreType.DMA,
        pltpu.SemaphoreType.DMA,
    ],
)
def _sc_gather(q_hbm, kv_hbm, qi_hbm, ki_hbm,
               sq_hbm, skv_hbm, idx_v, rows_a, rows_b, sem_a, sem_b):
    wid = _worker_id()
    for src, idx, dst in ((q_hbm, qi_hbm, sq_hbm),
                          (kv_hbm, ki_hbm, skv_hbm)):
        pltpu.sync_copy(idx.at[pl.ds(wid * CPW, CPW)], idx_v)

        def desc(c, buf, sem, src=src):
            return pltpu.make_async_copy(src.at[idx_v.at[c]], buf, sem)

        def store(c, buf, dst=dst):
            pltpu.sync_copy(buf, dst.at[pl.ds((wid * CPW + c) * CHUNK,
                                              CHUNK)])

        desc(0, rows_a, sem_a).start()

        def body(j, _):
            c0 = 2 * j
            desc(c0 + 1, rows_b, sem_b).start()
            desc(c0, rows_a, sem_a).wait()
            store(c0, rows_a)
            desc(c0 + 2, rows_a, sem_a).start()
            desc(c0 + 1, rows_b, sem_b).wait()
            store(c0 + 1, rows_b)
            return ()
        lax.fori_loop(0, CPW // 2 - 1, body, ())
        desc(CPW - 1, rows_b, sem_b).start()
        desc(CPW - 2, rows_a, sem_a).wait()
        store(CPW - 2, rows_a)
        desc(CPW - 1, rows_b, sem_b).wait()
        store(CPW - 1, rows_b)


@functools.partial(
    pl.kernel, mesh=_SC_MESH,
    out_type=jax.ShapeDtypeStruct((NROWS, DP), jnp.float32),
    scratch_types=[
        pltpu.VMEM((CPW, CHUNK), jnp.int32),
        pltpu.VMEM((CHUNK, DP), jnp.float32),
        pltpu.VMEM((CHUNK, DP), jnp.float32),
        pltpu.SemaphoreType.DMA,
        pltpu.SemaphoreType.DMA,
    ],
)
def _sc_scatter(so_hbm, qi_hbm, o_hbm, idx_v, rows_a, rows_b, sem_a, sem_b):
    wid = _worker_id()
    pltpu.sync_copy(qi_hbm.at[pl.ds(wid * CPW, CPW)], idx_v)

    def load(c, buf):
        pltpu.sync_copy(so_hbm.at[pl.ds((wid * CPW + c) * CHUNK, CHUNK)],
                        buf)

    def desc(c, buf, sem):
        return pltpu.make_async_copy(buf, o_hbm.at[idx_v.at[c]], sem)

    load(0, rows_a)
    desc(0, rows_a, sem_a).start()

    def body(j, _):
        c0 = 2 * j
        load(c0 + 1, rows_b)
        desc(c0 + 1, rows_b, sem_b).start()
        desc(c0, rows_a, sem_a).wait()
        load(c0 + 2, rows_a)
        desc(c0 + 2, rows_a, sem_a).start()
        desc(c0 + 1, rows_b, sem_b).wait()
        return ()
    lax.fori_loop(0, CPW // 2 - 1, body, ())
    load(CPW - 1, rows_b)
    desc(CPW - 1, rows_b, sem_b).start()
    desc(CPW - 2, rows_a, sem_a).wait()
    desc(CPW - 1, rows_b, sem_b).wait()


def _length_normalized(x, epsilon=1e-6):
    variance = jnp.mean(x ** 2, axis=-1, keepdims=True)
    return x / jnp.sqrt(variance + epsilon)


def kernel(hidden_states, Wq, Wk, Wv, Wo, R):
    q64 = jnp.einsum('bsd,dhf->bhsf', hidden_states, Wq).reshape(BH, S, DH)
    k64 = jnp.einsum('bsd,dhf->bhsf', hidden_states, Wk).reshape(BH, S, DH)
    v64 = jnp.einsum('bsd,dhf->bhsf', hidden_states, Wv).reshape(BH, S, DH)
    q_hash = jnp.einsum('nsf,f->ns', _length_normalized(q64), R)
    k_hash = jnp.einsum('nsf,f->ns', _length_normalized(k64), R)
    q = jnp.concatenate([q64, jnp.zeros_like(q64)], axis=-1)
    kv = jnp.concatenate([k64, v64], axis=-1)

    offs = jnp.arange(BH, dtype=jnp.int32)[:, None] * S
    q_idx = jnp.argsort(q_hash, axis=-1).astype(jnp.int32) + offs
    k_idx = jnp.argsort(k_hash, axis=-1).astype(jnp.int32) + offs
    qif = q_idx.reshape(NCHUNKS, CHUNK)
    kif = k_idx.reshape(NCHUNKS, CHUNK)

    sq, skv = _sc_gather(q.reshape(NROWS, DP), kv.reshape(NROWS, DP),
                         qif, kif)
    so = _chunked_attention(sq.reshape(BH, S, DP), skv.reshape(BH, S, DP),
                            q_idx, k_idx)
    o = _sc_scatter(so.reshape(NROWS, DP), qif)
    o = o.reshape(B, H, S, DP)
    Woz = jnp.concatenate([Wo, jnp.zeros_like(Wo)], axis=1)
    return jnp.einsum('bhsf,hfd->bsd', o, Woz)
